# baseline (speedup 1.0000x reference)
import jax
import jax.numpy as jnp
from jax import lax
from jax.experimental import pallas as pl
from jax.experimental.pallas import tpu as pltpu
from jax.experimental.pallas import tpu_sc as plsc

B = 1024
L_DOC = 200
L_Q = 50
LT = L_DOC + L_Q
D = 64
DP = 128
VOCAB = 1000000
NW = 32
NSLAB = -(-VOCAB // DP)
VPAD = NSLAB * DP
SFULL = NSLAB // NW * NW
TPW = B * LT // NW
CH = 80
NCH = TPW // CH
NBUF = 2
PEXT = 320


def _tr_kernel(tab_hbm, scr_hbm, in_v, out_v, *sems):
    isems = sems[:NBUF]
    osems = sems[NBUF:]
    wid = lax.axis_index("s") * 2 + lax.axis_index("c")

    def in_copy(s, b):
        return pltpu.make_async_copy(
            tab_hbm.at[:, pl.ds(s * DP, DP)], in_v.at[b], isems[b])

    def out_copy(s, b):
        return pltpu.make_async_copy(
            out_v.at[b], scr_hbm.at[pl.ds(s * DP, DP)], osems[b])

    rows_j = [lax.iota(jnp.int32, 16) + j * 16 for j in range(4)]

    def transpose(b):
        def tr_body(l, carry):
            colv = jnp.full((16,), l, jnp.int32)
            for j in range(4):
                out_v[b, l, pl.ds(j * 16, 16)] = plsc.load_gather(
                    in_v.at[b], [rows_j[j], colv])
            return carry
        lax.fori_loop(0, DP, tr_body, 0)

    for b in range(NBUF):
        in_copy(wid + b * NW, b).start()

    def louter(i0, carry):
        for b in range(NBUF):
            i = i0 * NBUF + b
            s = wid + i * NW
            in_copy(s, b).wait()

            @pl.when(i >= NBUF)
            def _wait_prev():
                out_copy(s - NBUF * NW, b).wait()

            transpose(b)

            @pl.when(i + NBUF < SFULL // NW)
            def _next_in():
                in_copy(s + NBUF * NW, b).start()

            out_copy(s, b).start()
        return carry

    lax.fori_loop(0, SFULL // NW // NBUF, louter, 0)
    for b in range(NBUF):
        out_copy(wid + (SFULL // NW - NBUF + b) * NW, b).wait()

    @pl.when(wid < NSLAB - SFULL)
    def _tail():
        s = SFULL + wid
        pltpu.sync_copy(tab_hbm.at[:, pl.ds(s * DP, DP)], in_v.at[0])
        transpose(0)
        pltpu.sync_copy(out_v.at[0], scr_hbm.at[pl.ds(s * DP, DP)])


def _emb_kernel(ids_hbm, pos_hbm, scr_hbm, out_hbm,
                idx_v, pos_v, rows_g, rows_o, *sems):
    gsems = sems[:NBUF]
    osems = sems[NBUF:]
    wid = lax.axis_index("s") * 2 + lax.axis_index("c")
    base = wid * TPW
    pltpu.sync_copy(ids_hbm.at[wid], idx_v)
    pltpu.sync_copy(pos_hbm, pos_v)

    def gather(c, b):
        return pltpu.make_async_copy(
            scr_hbm.at[idx_v.at[c]], rows_g.at[b], gsems[b])

    def out_copy(c, b):
        return pltpu.make_async_copy(
            rows_o.at[b], out_hbm.at[pl.ds(base + c * CH, CH)], osems[b])

    for b in range(NBUF):
        gather(b, b).start()

    def outer(c0, carry):
        for b in range(NBUF):
            c = c0 * NBUF + b
            gather(c, b).wait()

            @pl.when(c0 > 0)
            def _wait_prev():
                out_copy(c - NBUF, b).wait()

            poff = lax.rem(c * CH, LT)

            def add_body(r, carry2):
                for j in range(4):
                    sl = pl.ds(j * 16, 16)
                    rows_o[b, r, sl] = rows_g[b, r, sl] + pos_v[poff + r, sl]
                return carry2

            lax.fori_loop(0, CH, add_body, 0)

            @pl.when(c + NBUF < NCH)
            def _next_gather():
                gather(c + NBUF, b).start()

            out_copy(c, b).start()
        return carry

    lax.fori_loop(0, NCH // NBUF, outer, 0)
    for b in range(NBUF):
        out_copy(NCH - NBUF + b, b).wait()


def kernel(x1_ids, x2_ids, emb_table, pos_table):
    ids = jnp.concatenate([x1_ids, x2_ids], axis=1).astype(jnp.int32)
    ids_r = ids.reshape(NW, NCH, CH)
    pos_full = jnp.concatenate([pos_table[:L_DOC], pos_table[:L_Q]], axis=0)
    pos_ext = jnp.concatenate([pos_full, pos_full[: PEXT - LT]], axis=0)

    mesh = plsc.VectorSubcoreMesh(core_axis_name="c", subcore_axis_name="s")
    params = pltpu.CompilerParams(use_tc_tiling_on_sc=True, needs_layout_passes=False)

    scr = pl.kernel(
        _tr_kernel,
        out_type=jax.ShapeDtypeStruct((VPAD, DP), jnp.float32),
        mesh=mesh,
        compiler_params=params,
        scratch_types=[
            pltpu.VMEM((NBUF, D, DP), jnp.float32),
            pltpu.VMEM((NBUF, DP, DP), jnp.float32),
        ] + [pltpu.SemaphoreType.DMA] * (2 * NBUF),
    )(emb_table.T)

    out = pl.kernel(
        _emb_kernel,
        out_type=jax.ShapeDtypeStruct((B * LT, D), jnp.float32),
        mesh=mesh,
        compiler_params=params,
        scratch_types=[
            pltpu.VMEM((NCH, CH), jnp.int32),
            pltpu.VMEM((PEXT, D), jnp.float32),
            pltpu.VMEM((NBUF, CH, DP), jnp.float32),
            pltpu.VMEM((NBUF, CH, D), jnp.float32),
        ] + [pltpu.SemaphoreType.DMA] * (2 * NBUF),
    )(ids_r, pos_ext, scr)
    return out.reshape(B, LT, D)


# speedup vs baseline: 1.6102x; 1.6102x over previous
import jax
import jax.numpy as jnp
from jax import lax
from jax.experimental import pallas as pl
from jax.experimental.pallas import tpu as pltpu
from jax.experimental.pallas import tpu_sc as plsc

B = 1024
L_DOC = 200
L_Q = 50
LT = L_DOC + L_Q
D = 64
DP = 128
VOCAB = 1000000
NW = 32
NSLAB = -(-VOCAB // DP)
VPAD = NSLAB * DP
SFULL = NSLAB // NW * NW
TPW = B * LT // NW
CH = 80
NCH = TPW // CH
NBUF = 2
PEXT = 320


def _tr_kernel(tab_hbm, scr_hbm, in_v, out_v, *sems):
    isems = sems[:NBUF]
    osems = sems[NBUF:]
    wid = lax.axis_index("s") * 2 + lax.axis_index("c")

    def in_copy(s, b):
        return pltpu.make_async_copy(
            tab_hbm.at[:, pl.ds(s * DP, DP)], in_v.at[b], isems[b])

    def out_copy(s, b):
        return pltpu.make_async_copy(
            out_v.at[b], scr_hbm.at[pl.ds(s * DP, DP)], osems[b])

    rows_j = [lax.iota(jnp.int32, 16) + j * 16 for j in range(4)]

    def transpose(b):
        @plsc.parallel_loop(0, DP, step=8, unroll=4)
        def tr_body(l0):
            for dl in range(8):
                l = l0 + dl
                colv = jnp.full((16,), l, jnp.int32)
                for j in range(4):
                    out_v[b, l, pl.ds(j * 16, 16)] = plsc.load_gather(
                        in_v.at[b], [rows_j[j], colv])

    for b in range(NBUF):
        in_copy(wid + b * NW, b).start()

    def louter(i0, carry):
        for b in range(NBUF):
            i = i0 * NBUF + b
            s = wid + i * NW
            in_copy(s, b).wait()

            @pl.when(i >= NBUF)
            def _wait_prev():
                out_copy(s - NBUF * NW, b).wait()

            transpose(b)

            @pl.when(i + NBUF < SFULL // NW)
            def _next_in():
                in_copy(s + NBUF * NW, b).start()

            out_copy(s, b).start()
        return carry

    lax.fori_loop(0, SFULL // NW // NBUF, louter, 0)
    for b in range(NBUF):
        out_copy(wid + (SFULL // NW - NBUF + b) * NW, b).wait()

    @pl.when(wid < NSLAB - SFULL)
    def _tail():
        s = SFULL + wid
        pltpu.sync_copy(tab_hbm.at[:, pl.ds(s * DP, DP)], in_v.at[0])
        transpose(0)
        pltpu.sync_copy(out_v.at[0], scr_hbm.at[pl.ds(s * DP, DP)])


def _emb_kernel(ids_hbm, pos_hbm, scr_hbm, out_hbm,
                idx_v, pos_v, rows_g, rows_o, *sems):
    gsems = sems[:NBUF]
    osems = sems[NBUF:]
    wid = lax.axis_index("s") * 2 + lax.axis_index("c")
    base = wid * TPW
    pltpu.sync_copy(ids_hbm.at[wid], idx_v)
    pltpu.sync_copy(pos_hbm, pos_v)

    def gather(c, b):
        return pltpu.make_async_copy(
            scr_hbm.at[idx_v.at[c]], rows_g.at[b], gsems[b])

    def out_copy(c, b):
        return pltpu.make_async_copy(
            rows_o.at[b], out_hbm.at[pl.ds(base + c * CH, CH)], osems[b])

    for b in range(NBUF):
        gather(b, b).start()

    def outer(c0, carry):
        for b in range(NBUF):
            c = c0 * NBUF + b
            gather(c, b).wait()

            @pl.when(c0 > 0)
            def _wait_prev():
                out_copy(c - NBUF, b).wait()

            poff = lax.rem(c * CH, LT)

            def add_body(r, carry2):
                for j in range(4):
                    sl = pl.ds(j * 16, 16)
                    rows_o[b, r, sl] = rows_g[b, r, sl] + pos_v[poff + r, sl]
                return carry2

            lax.fori_loop(0, CH, add_body, 0)

            @pl.when(c + NBUF < NCH)
            def _next_gather():
                gather(c + NBUF, b).start()

            out_copy(c, b).start()
        return carry

    lax.fori_loop(0, NCH // NBUF, outer, 0)
    for b in range(NBUF):
        out_copy(NCH - NBUF + b, b).wait()


def kernel(x1_ids, x2_ids, emb_table, pos_table):
    ids = jnp.concatenate([x1_ids, x2_ids], axis=1).astype(jnp.int32)
    ids_r = ids.reshape(NW, NCH, CH)
    pos_full = jnp.concatenate([pos_table[:L_DOC], pos_table[:L_Q]], axis=0)
    pos_ext = jnp.concatenate([pos_full, pos_full[: PEXT - LT]], axis=0)

    mesh = plsc.VectorSubcoreMesh(core_axis_name="c", subcore_axis_name="s")
    params = pltpu.CompilerParams(use_tc_tiling_on_sc=True, needs_layout_passes=False)

    scr = pl.kernel(
        _tr_kernel,
        out_type=jax.ShapeDtypeStruct((VPAD, DP), jnp.float32),
        mesh=mesh,
        compiler_params=params,
        scratch_types=[
            pltpu.VMEM((NBUF, D, DP), jnp.float32),
            pltpu.VMEM((NBUF, DP, DP), jnp.float32),
        ] + [pltpu.SemaphoreType.DMA] * (2 * NBUF),
    )(emb_table.T)

    out = pl.kernel(
        _emb_kernel,
        out_type=jax.ShapeDtypeStruct((B * LT, D), jnp.float32),
        mesh=mesh,
        compiler_params=params,
        scratch_types=[
            pltpu.VMEM((NCH, CH), jnp.int32),
            pltpu.VMEM((PEXT, D), jnp.float32),
            pltpu.VMEM((NBUF, CH, DP), jnp.float32),
            pltpu.VMEM((NBUF, CH, D), jnp.float32),
        ] + [pltpu.SemaphoreType.DMA] * (2 * NBUF),
    )(ids_r, pos_ext, scr)
    return out.reshape(B, LT, D)


# speedup vs baseline: 1.6109x; 1.0004x over previous
import jax
import jax.numpy as jnp
from jax import lax
from jax.experimental import pallas as pl
from jax.experimental.pallas import tpu as pltpu
from jax.experimental.pallas import tpu_sc as plsc

B = 1024
L_DOC = 200
L_Q = 50
LT = L_DOC + L_Q
D = 64
DP = 128
VOCAB = 1000000
NW = 32
NSLAB = -(-VOCAB // DP)
VPAD = NSLAB * DP
SFULL = NSLAB // NW * NW
TPW = B * LT // NW
CH = 80
NCH = TPW // CH
NBUF = 2
PEXT = 320


def _tr_kernel(tab_hbm, scr_hbm, in_v, out_v, *sems):
    isems = sems[:NBUF]
    osems = sems[NBUF:]
    wid = lax.axis_index("s") * 2 + lax.axis_index("c")

    def in_copy(s, b):
        return pltpu.make_async_copy(
            tab_hbm.at[:, pl.ds(s * DP, DP)], in_v.at[b], isems[b])

    def out_copy(s, b):
        return pltpu.make_async_copy(
            out_v.at[b], scr_hbm.at[pl.ds(s * DP, DP)], osems[b])

    rows_j = [lax.iota(jnp.int32, 16) + j * 16 for j in range(4)]

    def transpose(b):
        @plsc.parallel_loop(0, DP, step=8, unroll=4)
        def tr_body(l0):
            for dl in range(8):
                l = l0 + dl
                colv = jnp.full((16,), l, jnp.int32)
                for j in range(4):
                    out_v[b, l, pl.ds(j * 16, 16)] = plsc.load_gather(
                        in_v.at[b], [rows_j[j], colv])

    for b in range(NBUF):
        in_copy(wid + b * NW, b).start()

    def louter(i0, carry):
        for b in range(NBUF):
            i = i0 * NBUF + b
            s = wid + i * NW
            in_copy(s, b).wait()

            @pl.when(i >= NBUF)
            def _wait_prev():
                out_copy(s - NBUF * NW, b).wait()

            transpose(b)

            @pl.when(i + NBUF < SFULL // NW)
            def _next_in():
                in_copy(s + NBUF * NW, b).start()

            out_copy(s, b).start()
        return carry

    lax.fori_loop(0, SFULL // NW // NBUF, louter, 0)
    for b in range(NBUF):
        out_copy(wid + (SFULL // NW - NBUF + b) * NW, b).wait()

    @pl.when(wid < NSLAB - SFULL)
    def _tail():
        s = SFULL + wid
        pltpu.sync_copy(tab_hbm.at[:, pl.ds(s * DP, DP)], in_v.at[0])
        transpose(0)
        pltpu.sync_copy(out_v.at[0], scr_hbm.at[pl.ds(s * DP, DP)])


def _emb_kernel(ids_hbm, pos_hbm, scr_hbm, out_hbm,
                idx_v, pos_v, rows_g, rows_o, *sems):
    gsems = sems[:NBUF]
    osems = sems[NBUF:]
    wid = lax.axis_index("s") * 2 + lax.axis_index("c")
    base = wid * TPW
    pltpu.sync_copy(ids_hbm.at[wid], idx_v)
    pltpu.sync_copy(pos_hbm, pos_v)

    def gather(c, b):
        return pltpu.make_async_copy(
            scr_hbm.at[idx_v.at[c]], rows_g.at[b], gsems[b])

    def out_copy(c, b):
        return pltpu.make_async_copy(
            rows_o.at[b], out_hbm.at[pl.ds(base + c * CH, CH)], osems[b])

    for b in range(NBUF):
        gather(b, b).start()

    def outer(c0, carry):
        for b in range(NBUF):
            c = c0 * NBUF + b
            gather(c, b).wait()

            @pl.when(c0 > 0)
            def _wait_prev():
                out_copy(c - NBUF, b).wait()

            poff = lax.rem(c * CH, LT)

            def add_body(r, carry2):
                for j in range(4):
                    sl = pl.ds(j * 16, 16)
                    rows_o[b, r, sl] = rows_g[b, r, sl] + pos_v[poff + r, sl]
                return carry2

            lax.fori_loop(0, CH, add_body, 0)

            @pl.when(c + NBUF < NCH)
            def _next_gather():
                gather(c + NBUF, b).start()

            out_copy(c, b).start()
        return carry

    lax.fori_loop(0, NCH // NBUF, outer, 0)
    for b in range(NBUF):
        out_copy(NCH - NBUF + b, b).wait()


def kernel(x1_ids, x2_ids, emb_table, pos_table):
    ids = jnp.concatenate([x1_ids, x2_ids], axis=1).astype(jnp.int32)
    ids_r = ids.reshape(NW, NCH, CH)
    pos_full = jnp.concatenate([pos_table[:L_DOC], pos_table[:L_Q]], axis=0)
    pos_ext = jnp.concatenate([pos_full, pos_full[: PEXT - LT]], axis=0)

    mesh = plsc.VectorSubcoreMesh(core_axis_name="c", subcore_axis_name="s")
    params = pltpu.CompilerParams(use_tc_tiling_on_sc=True, needs_layout_passes=False, disable_bounds_checks=True)

    scr = pl.kernel(
        _tr_kernel,
        out_type=jax.ShapeDtypeStruct((VPAD, DP), jnp.float32),
        mesh=mesh,
        compiler_params=params,
        scratch_types=[
            pltpu.VMEM((NBUF, D, DP), jnp.float32),
            pltpu.VMEM((NBUF, DP, DP), jnp.float32),
        ] + [pltpu.SemaphoreType.DMA] * (2 * NBUF),
    )(emb_table.T)

    out = pl.kernel(
        _emb_kernel,
        out_type=jax.ShapeDtypeStruct((B * LT, D), jnp.float32),
        mesh=mesh,
        compiler_params=params,
        scratch_types=[
            pltpu.VMEM((NCH, CH), jnp.int32),
            pltpu.VMEM((PEXT, D), jnp.float32),
            pltpu.VMEM((NBUF, CH, DP), jnp.float32),
            pltpu.VMEM((NBUF, CH, D), jnp.float32),
        ] + [pltpu.SemaphoreType.DMA] * (2 * NBUF),
    )(ids_r, pos_ext, scr)
    return out.reshape(B, LT, D)
